# baseline (speedup 1.0000x reference)
import functools

import jax
import jax.numpy as jnp
from jax import lax
from jax.experimental import pallas as pl
from jax.experimental.pallas import tpu as pltpu
from jax.experimental.pallas import tpu_sc as plsc

B = 16384
EMBED = 64
NC = 2
NS = 16
NW = NC * NS
BPW = B // NW
CH = 128
NCHUNK = BPW // CH

@functools.cache
def _make_sc_gather():
    mesh = plsc.VectorSubcoreMesh(core_axis_name="c", subcore_axis_name="s")

    @functools.partial(
        pl.kernel,
        out_type=(
            jax.ShapeDtypeStruct((B, EMBED), jnp.float32),
            jax.ShapeDtypeStruct((B, EMBED), jnp.float32),
        ),
        mesh=mesh,
        scratch_types=[
            pltpu.VMEM((NCHUNK, CH), jnp.int32),
            pltpu.VMEM((NCHUNK, CH), jnp.int32),
            pltpu.VMEM((BPW, EMBED), jnp.float32),
            pltpu.VMEM((BPW, EMBED), jnp.float32),
            pltpu.SemaphoreType.DMA,
        ],
        compiler_params=pltpu.CompilerParams(use_tc_tiling_on_sc=False),
    )
    def _sc_gather(uid_hbm, iid_hbm, utab_hbm, itab_hbm, uout_hbm, iout_hbm,
                   uidx_v, iidx_v, urows_v, irows_v, sem):
        wid = lax.axis_index("s") * NC + lax.axis_index("c")
        base = wid * BPW
        pltpu.sync_copy(uid_hbm.at[wid], uidx_v)
        pltpu.sync_copy(iid_hbm.at[wid], iidx_v)
        copies = []
        for j in range(NCHUNK):
            copies.append(pltpu.async_copy(
                utab_hbm.at[uidx_v.at[j]], urows_v.at[pl.ds(j * CH, CH)], sem))
            copies.append(pltpu.async_copy(
                itab_hbm.at[iidx_v.at[j]], irows_v.at[pl.ds(j * CH, CH)], sem))
        for c in copies:
            c.wait()
        pltpu.sync_copy(urows_v, uout_hbm.at[pl.ds(base, BPW)])
        pltpu.sync_copy(irows_v, iout_hbm.at[pl.ds(base, BPW)])

    return _sc_gather


def _mlp_body(ue_ref, ie_ref, feat_ref, w1u_ref, w1i_ref, w1f_ref,
              b1_ref, gamma_ref, beta_ref, w2_ref, b2_ref, out_ref):
    h = (jnp.dot(ue_ref[...], w1u_ref[...], preferred_element_type=jnp.float32)
         + jnp.dot(ie_ref[...], w1i_ref[...], preferred_element_type=jnp.float32)
         + jnp.dot(feat_ref[...], w1f_ref[...], preferred_element_type=jnp.float32)
         + b1_ref[...])
    mean = jnp.mean(h, axis=0, keepdims=True)
    d = h - mean
    var = jnp.mean(d * d, axis=0, keepdims=True)
    hn = d * lax.rsqrt(var + 1e-5) * gamma_ref[...] + beta_ref[...]
    hn = jnp.maximum(hn, 0.0)
    out_ref[...] = (jnp.sum(hn * w2_ref[...], axis=1, keepdims=True)
                    + b2_ref[...])


_mlp = pl.pallas_call(
    _mlp_body,
    out_shape=jax.ShapeDtypeStruct((B, 1), jnp.float32),
)


def kernel(user_id, item_id, users_info, items_info, user_table, item_table,
           W1, b1, gamma, beta, W2, b2):
    uid = user_id.reshape(NW, NCHUNK, CH)
    iid = item_id.reshape(NW, NCHUNK, CH)
    ue, ie = _make_sc_gather()(uid, iid, user_table, item_table)
    feats = jnp.concatenate([users_info, items_info], axis=1)
    return _mlp(ue, ie, feats,
                W1[:EMBED], W1[EMBED:2 * EMBED], W1[2 * EMBED:],
                b1.reshape(1, -1), gamma.reshape(1, -1), beta.reshape(1, -1),
                W2.reshape(1, -1), b2.reshape(1, 1))
